# baseline (device time: 92237 ns/iter reference)
import jax
import jax.numpy as jnp
from jax import lax
from jax.experimental import pallas as pl
from jax.experimental.pallas import tpu as pltpu

N_DEV = 4
N_WAVE = 4
KB = 4

RS_SCALE = 5.5 * 1536.0 / 127.0

AG_SCALE = 5.2 * 3072.0 / 127.0


def kernel(x, W1, W2):
    m, kdim = x.shape
    _, hdim = W1.shape
    _, n = W2.shape
    mc = m // N_DEV
    nw = n // N_WAVE
    hb = hdim // KB
    kb = hdim // KB

    def body(
        x_hbm, w1_hbm, w2_hbm, out_ref,
        xstage, xb, w1stage, w1blk, w2stage, w2bf, h_ref,
        rs_send, rs_recv, ag_send, ag_recv,
        ld_sems, rs_send_sems, rs_recv_sems, ag_send_sems, ag_recv_sems,
    ):
        my = lax.axis_index("i")

        barrier = pltpu.get_barrier_semaphore()
        for j in range(1, N_DEV):
            pl.semaphore_signal(
                barrier, inc=1, device_id=(lax.rem(my + j, N_DEV),),
                device_id_type=pl.DeviceIdType.MESH,
            )
        pl.semaphore_wait(barrier, N_DEV - 1)

        ld_w1 = pltpu.make_async_copy(
            w1_hbm.at[:, pl.ds(0, hb)], w1stage.at[0], ld_sems.at[0]
        )
        ld_w1.start()
        ld_xs = []
        for c in range(2):
            ld_x = pltpu.make_async_copy(
                x_hbm.at[pl.ds(c * mc, mc), :], xstage.at[c], ld_sems.at[2 + c]
            )
            ld_x.start()
            ld_xs.append(ld_x)
        ld_w1.wait()
        w1blk[0, :, :] = w1stage[0, :, :].astype(jnp.bfloat16)
        for c in range(N_DEV):
            ld_xs[c].wait()
            xb[pl.ds(c * mc, mc), :] = xstage[c % 2, :, :].astype(jnp.bfloat16)
            if c + 2 < N_DEV:
                ld_x = pltpu.make_async_copy(
                    x_hbm.at[pl.ds((c + 2) * mc, mc), :],
                    xstage.at[c % 2],
                    ld_sems.at[2 + c % 2],
                )
                ld_x.start()
                ld_xs.append(ld_x)

        for b in range(KB):
            if b + 1 < KB:
                ld_w1 = pltpu.make_async_copy(
                    w1_hbm.at[:, pl.ds((b + 1) * hb, hb)],
                    w1stage.at[(b + 1) % 2],
                    ld_sems.at[0],
                )
                ld_w1.start()
            ld_w2 = pltpu.make_async_copy(
                w2_hbm.at[pl.ds(b * kb, kb), :], w2stage, ld_sems.at[1]
            )
            ld_w2.start()
            for c in range(N_DEV):
                h_ref[pl.ds(c * mc, mc), pl.ds(b * hb, hb)] = jnp.maximum(
                    jnp.dot(
                        xb[pl.ds(c * mc, mc), :], w1blk[b % 2, :, :],
                        preferred_element_type=jnp.float32,
                    ),
                    0.0,
                ).astype(jnp.bfloat16)
            if b + 1 < KB:
                ld_w1.wait()
                w1blk[(b + 1) % 2, :, :] = (
                    w1stage[(b + 1) % 2, :, :].astype(jnp.bfloat16)
                )
            ld_w2.wait()
            w2bf[pl.ds(b * kb, kb), :] = w2stage[...].astype(jnp.bfloat16)

        sends = []

        def rs_wave(w):
            for j in range(1, N_DEV):
                c = lax.rem(my + j, N_DEV)
                p = jnp.dot(
                    h_ref[pl.ds(c * mc, mc), :], w2bf[:, pl.ds(w * nw, nw)],
                    preferred_element_type=jnp.float32,
                )
                q = jnp.clip(jnp.round(p * (1.0 / RS_SCALE)), -127.0, 127.0)
                rs_send[w, j - 1, :, :] = q.astype(jnp.int8)
                rdma = pltpu.make_async_remote_copy(
                    src_ref=rs_send.at[w, j - 1],
                    dst_ref=rs_recv.at[w, N_DEV - 1 - j],
                    send_sem=rs_send_sems.at[w, j - 1],
                    recv_sem=rs_recv_sems.at[w, N_DEV - 1 - j],
                    device_id=(c,),
                    device_id_type=pl.DeviceIdType.MESH,
                )
                rdma.start()
                sends.append(rdma)

            own = jnp.dot(
                h_ref[pl.ds(my * mc, mc), :], w2bf[:, pl.ds(w * nw, nw)],
                preferred_element_type=jnp.float32,
            )

            for k in range(N_DEV - 1):
                recv = pltpu.make_async_remote_copy(
                    src_ref=rs_send.at[w, 0],
                    dst_ref=rs_recv.at[w, k],
                    send_sem=rs_send_sems.at[w, 0],
                    recv_sem=rs_recv_sems.at[w, k],
                    device_id=(my,),
                    device_id_type=pl.DeviceIdType.MESH,
                )
                recv.wait_recv()
                own = own + rs_recv[w, k, :, :].astype(jnp.float32) * RS_SCALE

            qown = jnp.clip(jnp.round(own * (1.0 / AG_SCALE)), -127.0, 127.0)
            ag_send[w, :, :] = qown.astype(jnp.int8)
            for j in range(1, N_DEV):
                rdma = pltpu.make_async_remote_copy(
                    src_ref=ag_send.at[w],
                    dst_ref=ag_recv.at[w, N_DEV - 1 - j],
                    send_sem=ag_send_sems.at[w, j - 1],
                    recv_sem=ag_recv_sems.at[w, N_DEV - 1 - j],
                    device_id=(lax.rem(my + j, N_DEV),),
                    device_id_type=pl.DeviceIdType.MESH,
                )
                rdma.start()
                sends.append(rdma)
            out_ref[pl.ds(my * mc, mc), pl.ds(w * nw, nw)] = own.astype(jnp.bfloat16)

        def ag_drain(w):
            for k in range(1, N_DEV):
                recv = pltpu.make_async_remote_copy(
                    src_ref=ag_send.at[w],
                    dst_ref=ag_recv.at[w, k - 1],
                    send_sem=ag_send_sems.at[w, 0],
                    recv_sem=ag_recv_sems.at[w, k - 1],
                    device_id=(my,),
                    device_id_type=pl.DeviceIdType.MESH,
                )
                recv.wait_recv()
                c = lax.rem(my + k, N_DEV)
                out_ref[pl.ds(c * mc, mc), pl.ds(w * nw, nw)] = (
                    ag_recv[w, k - 1, :, :].astype(jnp.float32) * AG_SCALE
                ).astype(jnp.bfloat16)

        for w in range(N_WAVE):
            rs_wave(w)
        for w in range(N_WAVE):
            ag_drain(w)

        for rdma in sends:
            rdma.wait_send()

    return pl.pallas_call(
        body,
        out_shape=jax.ShapeDtypeStruct((m, n), jnp.bfloat16),
        in_specs=[pl.BlockSpec(memory_space=pl.ANY)] * 3,
        out_specs=pl.BlockSpec(memory_space=pltpu.VMEM),
        scratch_shapes=[
            pltpu.VMEM((2, mc, kdim), jnp.float32),
            pltpu.VMEM((m, kdim), jnp.bfloat16),
            pltpu.VMEM((2, kdim, hb), jnp.float32),
            pltpu.VMEM((2, kdim, hb), jnp.bfloat16),
            pltpu.VMEM((kb, n), jnp.float32),
            pltpu.VMEM((hdim, n), jnp.bfloat16),
            pltpu.VMEM((m, hdim), jnp.bfloat16),
            pltpu.VMEM((N_WAVE, N_DEV - 1, mc, nw), jnp.int8),
            pltpu.VMEM((N_WAVE, N_DEV - 1, mc, nw), jnp.int8),
            pltpu.VMEM((N_WAVE, mc, nw), jnp.int8),
            pltpu.VMEM((N_WAVE, N_DEV - 1, mc, nw), jnp.int8),
            pltpu.SemaphoreType.DMA((4,)),
            pltpu.SemaphoreType.DMA((N_WAVE, N_DEV - 1)),
            pltpu.SemaphoreType.DMA((N_WAVE, N_DEV - 1)),
            pltpu.SemaphoreType.DMA((N_WAVE, N_DEV - 1)),
            pltpu.SemaphoreType.DMA((N_WAVE, N_DEV - 1)),
        ],
        compiler_params=pltpu.CompilerParams(
            collective_id=0, vmem_limit_bytes=128 * 1024 * 1024
        ),
    )(x, W1, W2)


# device time: 90620 ns/iter; 1.0178x vs baseline; 1.0178x over previous
import jax
import jax.numpy as jnp
from jax import lax
from jax.experimental import pallas as pl
from jax.experimental.pallas import tpu as pltpu

N_DEV = 4
N_WAVE = 2
KB = 4

RS_SCALE = 5.5 * 1536.0 / 127.0

AG_SCALE = 5.2 * 3072.0 / 127.0


def kernel(x, W1, W2):
    m, kdim = x.shape
    _, hdim = W1.shape
    _, n = W2.shape
    mc = m // N_DEV
    nw = n // N_WAVE
    hb = hdim // KB
    kb = hdim // KB

    def body(
        x_hbm, w1_hbm, w2_hbm, out_ref,
        xstage, xb, w1stage, w1blk, w2stage, w2bf, h_ref,
        rs_send, rs_recv, ag_send, ag_recv,
        ld_sems, rs_send_sems, rs_recv_sems, ag_send_sems, ag_recv_sems,
    ):
        my = lax.axis_index("i")

        barrier = pltpu.get_barrier_semaphore()
        for j in range(1, N_DEV):
            pl.semaphore_signal(
                barrier, inc=1, device_id=(lax.rem(my + j, N_DEV),),
                device_id_type=pl.DeviceIdType.MESH,
            )
        pl.semaphore_wait(barrier, N_DEV - 1)

        ld_w1 = pltpu.make_async_copy(
            w1_hbm.at[:, pl.ds(0, hb)], w1stage.at[0], ld_sems.at[0]
        )
        ld_w1.start()
        ld_xs = []
        for c in range(2):
            ld_x = pltpu.make_async_copy(
                x_hbm.at[pl.ds(c * mc, mc), :], xstage.at[c], ld_sems.at[2 + c]
            )
            ld_x.start()
            ld_xs.append(ld_x)
        ld_w1.wait()
        w1blk[0, :, :] = w1stage[0, :, :].astype(jnp.bfloat16)
        for c in range(N_DEV):
            ld_xs[c].wait()
            xb[pl.ds(c * mc, mc), :] = xstage[c % 2, :, :].astype(jnp.bfloat16)
            if c + 2 < N_DEV:
                ld_x = pltpu.make_async_copy(
                    x_hbm.at[pl.ds((c + 2) * mc, mc), :],
                    xstage.at[c % 2],
                    ld_sems.at[2 + c % 2],
                )
                ld_x.start()
                ld_xs.append(ld_x)

        for b in range(KB):
            if b + 1 < KB:
                ld_w1 = pltpu.make_async_copy(
                    w1_hbm.at[:, pl.ds((b + 1) * hb, hb)],
                    w1stage.at[(b + 1) % 2],
                    ld_sems.at[0],
                )
                ld_w1.start()
            ld_w2 = pltpu.make_async_copy(
                w2_hbm.at[pl.ds(b * kb, kb), :], w2stage, ld_sems.at[1]
            )
            ld_w2.start()
            for c in range(N_DEV):
                h_ref[pl.ds(c * mc, mc), pl.ds(b * hb, hb)] = jnp.maximum(
                    jnp.dot(
                        xb[pl.ds(c * mc, mc), :], w1blk[b % 2, :, :],
                        preferred_element_type=jnp.float32,
                    ),
                    0.0,
                ).astype(jnp.bfloat16)
            if b + 1 < KB:
                ld_w1.wait()
                w1blk[(b + 1) % 2, :, :] = (
                    w1stage[(b + 1) % 2, :, :].astype(jnp.bfloat16)
                )
            ld_w2.wait()
            w2bf[pl.ds(b * kb, kb), :] = w2stage[...].astype(jnp.bfloat16)

        sends = []

        def rs_wave(w):
            for j in range(1, N_DEV):
                c = lax.rem(my + j, N_DEV)
                p = jnp.dot(
                    h_ref[pl.ds(c * mc, mc), :], w2bf[:, pl.ds(w * nw, nw)],
                    preferred_element_type=jnp.float32,
                )
                q = jnp.clip(jnp.round(p * (1.0 / RS_SCALE)), -127.0, 127.0)
                rs_send[w, j - 1, :, :] = q.astype(jnp.int8)
                rdma = pltpu.make_async_remote_copy(
                    src_ref=rs_send.at[w, j - 1],
                    dst_ref=rs_recv.at[w, N_DEV - 1 - j],
                    send_sem=rs_send_sems.at[w, j - 1],
                    recv_sem=rs_recv_sems.at[w, N_DEV - 1 - j],
                    device_id=(c,),
                    device_id_type=pl.DeviceIdType.MESH,
                )
                rdma.start()
                sends.append(rdma)

            own = jnp.dot(
                h_ref[pl.ds(my * mc, mc), :], w2bf[:, pl.ds(w * nw, nw)],
                preferred_element_type=jnp.float32,
            )

            for k in range(N_DEV - 1):
                recv = pltpu.make_async_remote_copy(
                    src_ref=rs_send.at[w, 0],
                    dst_ref=rs_recv.at[w, k],
                    send_sem=rs_send_sems.at[w, 0],
                    recv_sem=rs_recv_sems.at[w, k],
                    device_id=(my,),
                    device_id_type=pl.DeviceIdType.MESH,
                )
                recv.wait_recv()
                own = own + rs_recv[w, k, :, :].astype(jnp.float32) * RS_SCALE

            qown = jnp.clip(jnp.round(own * (1.0 / AG_SCALE)), -127.0, 127.0)
            ag_send[w, :, :] = qown.astype(jnp.int8)
            for j in range(1, N_DEV):
                rdma = pltpu.make_async_remote_copy(
                    src_ref=ag_send.at[w],
                    dst_ref=ag_recv.at[w, N_DEV - 1 - j],
                    send_sem=ag_send_sems.at[w, j - 1],
                    recv_sem=ag_recv_sems.at[w, N_DEV - 1 - j],
                    device_id=(lax.rem(my + j, N_DEV),),
                    device_id_type=pl.DeviceIdType.MESH,
                )
                rdma.start()
                sends.append(rdma)
            out_ref[pl.ds(my * mc, mc), pl.ds(w * nw, nw)] = own.astype(jnp.bfloat16)

        def ag_drain(w):
            for k in range(1, N_DEV):
                recv = pltpu.make_async_remote_copy(
                    src_ref=ag_send.at[w],
                    dst_ref=ag_recv.at[w, k - 1],
                    send_sem=ag_send_sems.at[w, 0],
                    recv_sem=ag_recv_sems.at[w, k - 1],
                    device_id=(my,),
                    device_id_type=pl.DeviceIdType.MESH,
                )
                recv.wait_recv()
                c = lax.rem(my + k, N_DEV)
                out_ref[pl.ds(c * mc, mc), pl.ds(w * nw, nw)] = (
                    ag_recv[w, k - 1, :, :].astype(jnp.float32) * AG_SCALE
                ).astype(jnp.bfloat16)

        for w in range(N_WAVE):
            rs_wave(w)
        for w in range(N_WAVE):
            ag_drain(w)

        for rdma in sends:
            rdma.wait_send()

    return pl.pallas_call(
        body,
        out_shape=jax.ShapeDtypeStruct((m, n), jnp.bfloat16),
        in_specs=[pl.BlockSpec(memory_space=pl.ANY)] * 3,
        out_specs=pl.BlockSpec(memory_space=pltpu.VMEM),
        scratch_shapes=[
            pltpu.VMEM((2, mc, kdim), jnp.float32),
            pltpu.VMEM((m, kdim), jnp.bfloat16),
            pltpu.VMEM((2, kdim, hb), jnp.float32),
            pltpu.VMEM((2, kdim, hb), jnp.bfloat16),
            pltpu.VMEM((kb, n), jnp.float32),
            pltpu.VMEM((hdim, n), jnp.bfloat16),
            pltpu.VMEM((m, hdim), jnp.bfloat16),
            pltpu.VMEM((N_WAVE, N_DEV - 1, mc, nw), jnp.int8),
            pltpu.VMEM((N_WAVE, N_DEV - 1, mc, nw), jnp.int8),
            pltpu.VMEM((N_WAVE, mc, nw), jnp.int8),
            pltpu.VMEM((N_WAVE, N_DEV - 1, mc, nw), jnp.int8),
            pltpu.SemaphoreType.DMA((4,)),
            pltpu.SemaphoreType.DMA((N_WAVE, N_DEV - 1)),
            pltpu.SemaphoreType.DMA((N_WAVE, N_DEV - 1)),
            pltpu.SemaphoreType.DMA((N_WAVE, N_DEV - 1)),
            pltpu.SemaphoreType.DMA((N_WAVE, N_DEV - 1)),
        ],
        compiler_params=pltpu.CompilerParams(
            collective_id=0, vmem_limit_bytes=128 * 1024 * 1024
        ),
    )(x, W1, W2)
